# baseline (speedup 1.0000x reference)
import functools

import jax
import jax.numpy as jnp
from jax import lax
from jax.experimental import pallas as pl
from jax.experimental.pallas import tpu as pltpu
from jax.experimental.pallas import tpu_sc as plsc

N_ATOMS = 10000
N_EDGES = 320000
D = 128
NC = 2
NS = 16
NW = NC * NS
EDGES_PER_W = N_EDGES // NW
E = 64
NCH = EDGES_PER_W // E
ET = EDGES_PER_W - NCH * E
NB = 3
NBM = 6
ROWS_PER_TILE = 632
ROWS_LAST = N_ATOMS - 15 * ROWS_PER_TILE
MU = 4
RB_CH = [(t * E, E) for t in range(9)] + [(9 * E, ROWS_PER_TILE - 9 * E)]
RB_CH_LAST = [(t * E, E) for t in range(8)] + [(8 * E, ROWS_LAST - 8 * E)]


def _mm1_body(x_ref, w_ref, o_ref):
    o_ref[...] = jnp.dot(x_ref[...], w_ref[...],
                         preferred_element_type=jnp.float32)


def _mm2_body(p_ref, w_ref, b_ref, o_ref):
    s = p_ref[0:N_ATOMS, :] + p_ref[N_ATOMS:2 * N_ATOMS, :]
    o_ref[...] = jnp.dot(s, w_ref[...],
                         preferred_element_type=jnp.float32) + b_ref[...]


def _make_sc_kernel():
    mesh = plsc.VectorSubcoreMesh(core_axis_name="c", subcore_axis_name="s")

    @functools.partial(
        pl.kernel,
        mesh=mesh,
        out_type=jax.ShapeDtypeStruct((2 * N_ATOMS, D), jnp.float32),
        scratch_types=[
            [pltpu.VMEM((2, E), jnp.int32) for _ in range(NBM)],
            [pltpu.VMEM((E, D), jnp.float32) for _ in range(NB)],
            [pltpu.VMEM((E, D), jnp.float32) for _ in range(NB)],
            pltpu.VMEM((2, ET), jnp.int32),
            pltpu.VMEM_SHARED((N_ATOMS, D), jnp.float32),
            [pltpu.SemaphoreType.DMA for _ in range(NB)],
            [pltpu.SemaphoreType.DMA for _ in range(NB)],
            [pltpu.SemaphoreType.DMA for _ in range(NB)],
        ],
    )
    def sc_fn(f_hbm, w_hbm, meta_hbm, tailm_hbm, out_hbm,
              meta_v, rows_v, wv, tailm_v, conv_sh,
              sem_in, sem_g, sem_sc):
        c = lax.axis_index("c")
        s = lax.axis_index("s")
        wid = s * NC + c
        ebase = wid * EDGES_PER_W
        cbase = wid * NCH

        def zrow(r, carry):
            for j in range(D // 16):
                wv[0][r, pl.ds(j * 16, 16)] = jnp.zeros((16,), jnp.float32)
            return carry
        lax.fori_loop(0, E, zrow, 0)

        @pl.when(s < NS - 1)
        def _():
            for off, ln in RB_CH:
                pltpu.sync_copy(
                    wv[0].at[pl.ds(0, ln)],
                    conv_sh.at[pl.ds(s * ROWS_PER_TILE + off, ln)])

        @pl.when(s == NS - 1)
        def _():
            for off, ln in RB_CH_LAST:
                pltpu.sync_copy(
                    wv[0].at[pl.ds(0, ln)],
                    conv_sh.at[pl.ds(s * ROWS_PER_TILE + off, ln)])
        plsc.subcore_barrier()

        def start_inputs(k, b3, b6):
            pltpu.async_copy(meta_hbm.at[cbase + k], meta_v[b6], sem_in[b3])
            pltpu.async_copy(w_hbm.at[pl.ds(ebase + k * E, E)], wv[b3],
                             sem_in[b3])

        def wait_inputs(b3, b6):
            pltpu.make_async_copy(meta_hbm.at[0], meta_v[b6],
                                  sem_in[b3]).wait()
            pltpu.make_async_copy(w_hbm.at[pl.ds(0, E)], wv[b3],
                                  sem_in[b3]).wait()

        def start_gather(b3, b6):
            pltpu.async_copy(f_hbm.at[meta_v[b6].at[0]], rows_v[b3],
                             sem_g[b3])

        def wait_gather(b3, b6):
            pltpu.make_async_copy(f_hbm.at[meta_v[b6].at[0]], rows_v[b3],
                                  sem_g[b3]).wait()

        def mul(b3):
            def mrow(it, cc):
                for u in range(MU):
                    r = MU * it + u
                    for j in range(D // 16):
                        sl = pl.ds(j * 16, 16)
                        rows_v[b3][r, sl] = rows_v[b3][r, sl] * wv[b3][r, sl]
                return cc
            lax.fori_loop(0, E // MU, mrow, 0)

        def start_scatter(b3, b6):
            pltpu.async_copy(rows_v[b3], conv_sh.at[meta_v[b6].at[1]],
                             sem_sc[b3], add=True)

        def wait_scatter(b3, b6):
            pltpu.make_async_copy(rows_v[b3], conv_sh.at[meta_v[b6].at[1]],
                                  sem_sc[b3]).wait()

        def step(st):
            if st >= 2:
                wait_scatter((st - 2) % NB, (st - 2) % NBM)
            if st + 2 < NCH:
                start_inputs(st + 2, (st + 2) % NB, (st + 2) % NBM)
            if st + 1 < NCH:
                wait_inputs((st + 1) % NB, (st + 1) % NBM)
                start_gather((st + 1) % NB, (st + 1) % NBM)
            wait_gather(st % NB, st % NBM)
            mul(st % NB)
            start_scatter(st % NB, st % NBM)

        start_inputs(0, 0, 0)
        start_inputs(1, 1, 1)
        wait_inputs(0, 0)
        start_gather(0, 0)
        for st in range(NBM):
            step(st)

        def step6(i, carry):
            for u in range(NBM):
                st = NBM * i + u
                wait_scatter((u - 2) % NB, (u - 2) % NBM)
                start_inputs(st + 2, (u + 2) % NB, (u + 2) % NBM)
                wait_inputs((u + 1) % NB, (u + 1) % NBM)
                start_gather((u + 1) % NB, (u + 1) % NBM)
                wait_gather(u % NB, u % NBM)
                mul(u % NB)
                start_scatter(u % NB, u % NBM)
            return carry
        lax.fori_loop(1, (NCH - NBM) // NBM, step6, 0)

        for st in range(NCH - NBM, NCH):
            step(st)
        for st in range(NCH - 2, NCH):
            wait_scatter(st % NB, st % NBM)

        pltpu.sync_copy(tailm_hbm.at[wid], tailm_v)
        pltpu.sync_copy(w_hbm.at[pl.ds(ebase + NCH * E, ET)],
                        wv[0].at[pl.ds(0, ET)])
        pltpu.async_copy(f_hbm.at[tailm_v.at[0]],
                         rows_v[0].at[pl.ds(0, ET)], sem_g[0]).wait()

        def trow(r, cc):
            for j in range(D // 16):
                sl = pl.ds(j * 16, 16)
                rows_v[0][r, sl] = rows_v[0][r, sl] * wv[0][r, sl]
            return cc
        lax.fori_loop(0, ET, trow, 0)
        pltpu.sync_copy(rows_v[0].at[pl.ds(0, ET)],
                        conv_sh.at[tailm_v.at[1]], add=True)

        plsc.subcore_barrier()

        @pl.when(s < NS - 1)
        def _():
            for off, ln in RB_CH:
                src_off = s * ROWS_PER_TILE + off
                pltpu.sync_copy(conv_sh.at[pl.ds(src_off, ln)],
                                wv[0].at[pl.ds(0, ln)])
                pltpu.sync_copy(wv[0].at[pl.ds(0, ln)],
                                out_hbm.at[pl.ds(c * N_ATOMS + src_off, ln)])

        @pl.when(s == NS - 1)
        def _():
            for off, ln in RB_CH_LAST:
                src_off = s * ROWS_PER_TILE + off
                pltpu.sync_copy(conv_sh.at[pl.ds(src_off, ln)],
                                wv[0].at[pl.ds(0, ln)])
                pltpu.sync_copy(wv[0].at[pl.ds(0, ln)],
                                out_hbm.at[pl.ds(c * N_ATOMS + src_off, ln)])

    return sc_fn


_sc_kernel = _make_sc_kernel()


def kernel(x, w, seg_i, idx_j, W_in2fac, W_fac2out, b_fac2out):
    seg = seg_i.astype(jnp.int32).reshape(NW, EDGES_PER_W)
    idx = idx_j.astype(jnp.int32).reshape(NW, EDGES_PER_W)
    meta = jnp.stack(
        [idx[:, :NCH * E].reshape(NW, NCH, E),
         seg[:, :NCH * E].reshape(NW, NCH, E)], axis=2
    ).reshape(NW * NCH, 2, E)
    tailm = jnp.stack([idx[:, NCH * E:], seg[:, NCH * E:]], axis=1)

    f = pl.pallas_call(
        _mm1_body,
        out_shape=jax.ShapeDtypeStruct((N_ATOMS, D), jnp.float32),
    )(x, W_in2fac)

    parts = _sc_kernel(f, w, meta, tailm)

    y = pl.pallas_call(
        _mm2_body,
        out_shape=jax.ShapeDtypeStruct((N_ATOMS, D), jnp.float32),
    )(parts, W_fac2out, b_fac2out.reshape(1, D))
    return y


# speedup vs baseline: 1.0461x; 1.0461x over previous
import functools

import jax
import jax.numpy as jnp
from jax import lax
from jax.experimental import pallas as pl
from jax.experimental.pallas import tpu as pltpu
from jax.experimental.pallas import tpu_sc as plsc

N_ATOMS = 10000
N_EDGES = 320000
D = 128
NC = 2
NS = 16
NW = NC * NS
EDGES_PER_W = N_EDGES // NW
E = 64
NCH = EDGES_PER_W // E
ET = EDGES_PER_W - NCH * E
NB = 3
NBM = 6
ROWS_PER_TILE = 632
ROWS_LAST = N_ATOMS - 15 * ROWS_PER_TILE
MU = 4
RB_CH = [(t * E, E) for t in range(9)] + [(9 * E, ROWS_PER_TILE - 9 * E)]
RB_CH_LAST = [(t * E, E) for t in range(8)] + [(8 * E, ROWS_LAST - 8 * E)]


def _mm1_body(x_ref, w_ref, o_ref):
    o_ref[...] = jnp.dot(x_ref[...], w_ref[...],
                         preferred_element_type=jnp.float32)


def _mm2_body(p_ref, w_ref, b_ref, o_ref):
    s = p_ref[0:N_ATOMS, :] + p_ref[N_ATOMS:2 * N_ATOMS, :]
    o_ref[...] = jnp.dot(s, w_ref[...],
                         preferred_element_type=jnp.float32) + b_ref[...]


def _make_sc_kernel():
    mesh = plsc.VectorSubcoreMesh(core_axis_name="c", subcore_axis_name="s")

    @functools.partial(
        pl.kernel,
        mesh=mesh,
        out_type=jax.ShapeDtypeStruct((2 * N_ATOMS, D), jnp.float32),
        scratch_types=[
            [pltpu.VMEM((E,), jnp.int32) for _ in range(NBM)],
            [pltpu.VMEM((E,), jnp.int32) for _ in range(NBM)],
            [pltpu.VMEM((E, D), jnp.float32) for _ in range(NB)],
            [pltpu.VMEM((E, D), jnp.float32) for _ in range(NB)],
            pltpu.VMEM((2, ET), jnp.int32),
            pltpu.VMEM_SHARED((N_ATOMS, D), jnp.float32),
            [pltpu.SemaphoreType.DMA for _ in range(NB)],
            [pltpu.SemaphoreType.DMA for _ in range(NB)],
            [pltpu.SemaphoreType.DMA for _ in range(NB)],
        ],
    )
    def sc_fn(f_hbm, w_hbm, seg_hbm, idx_hbm, tailm_hbm, out_hbm,
              idx_v, seg_v, rows_v, wv, tailm_v, conv_sh,
              sem_in, sem_g, sem_sc):
        c = lax.axis_index("c")
        s = lax.axis_index("s")
        wid = s * NC + c
        ebase = wid * EDGES_PER_W

        def zrow(r, carry):
            for j in range(D // 16):
                wv[0][r, pl.ds(j * 16, 16)] = jnp.zeros((16,), jnp.float32)
            return carry
        lax.fori_loop(0, E, zrow, 0)

        @pl.when(s < NS - 1)
        def _():
            for off, ln in RB_CH:
                pltpu.sync_copy(
                    wv[0].at[pl.ds(0, ln)],
                    conv_sh.at[pl.ds(s * ROWS_PER_TILE + off, ln)])

        @pl.when(s == NS - 1)
        def _():
            for off, ln in RB_CH_LAST:
                pltpu.sync_copy(
                    wv[0].at[pl.ds(0, ln)],
                    conv_sh.at[pl.ds(s * ROWS_PER_TILE + off, ln)])
        plsc.subcore_barrier()

        def start_inputs(k, b3, b6):
            base = ebase + k * E
            pltpu.async_copy(idx_hbm.at[pl.ds(base, E)], idx_v[b6],
                             sem_in[b3])
            pltpu.async_copy(seg_hbm.at[pl.ds(base, E)], seg_v[b6],
                             sem_in[b3])
            pltpu.async_copy(w_hbm.at[pl.ds(base, E)], wv[b3], sem_in[b3])

        def wait_inputs(b3, b6):
            pltpu.make_async_copy(idx_hbm.at[pl.ds(0, E)], idx_v[b6],
                                  sem_in[b3]).wait()
            pltpu.make_async_copy(seg_hbm.at[pl.ds(0, E)], seg_v[b6],
                                  sem_in[b3]).wait()
            pltpu.make_async_copy(w_hbm.at[pl.ds(0, E)], wv[b3],
                                  sem_in[b3]).wait()

        def start_gather(b3, b6):
            pltpu.async_copy(f_hbm.at[idx_v[b6]], rows_v[b3], sem_g[b3])

        def wait_gather(b3, b6):
            pltpu.make_async_copy(f_hbm.at[idx_v[b6]], rows_v[b3],
                                  sem_g[b3]).wait()

        def mul(b3):
            def mrow(it, cc):
                for u in range(MU):
                    r = MU * it + u
                    for j in range(D // 16):
                        sl = pl.ds(j * 16, 16)
                        rows_v[b3][r, sl] = rows_v[b3][r, sl] * wv[b3][r, sl]
                return cc
            lax.fori_loop(0, E // MU, mrow, 0)

        def start_scatter(b3, b6):
            pltpu.async_copy(rows_v[b3], conv_sh.at[seg_v[b6]],
                             sem_sc[b3], add=True)

        def wait_scatter(b3, b6):
            pltpu.make_async_copy(rows_v[b3], conv_sh.at[seg_v[b6]],
                                  sem_sc[b3]).wait()

        def step(st):
            if st >= 2:
                wait_scatter((st - 2) % NB, (st - 2) % NBM)
            if st + 2 < NCH:
                start_inputs(st + 2, (st + 2) % NB, (st + 2) % NBM)
            if st + 1 < NCH:
                wait_inputs((st + 1) % NB, (st + 1) % NBM)
                start_gather((st + 1) % NB, (st + 1) % NBM)
            wait_gather(st % NB, st % NBM)
            mul(st % NB)
            start_scatter(st % NB, st % NBM)

        start_inputs(0, 0, 0)
        start_inputs(1, 1, 1)
        wait_inputs(0, 0)
        start_gather(0, 0)
        for st in range(NBM):
            step(st)

        def step6(i, carry):
            for u in range(NBM):
                st = NBM * i + u
                wait_scatter((u - 2) % NB, (u - 2) % NBM)
                start_inputs(st + 2, (u + 2) % NB, (u + 2) % NBM)
                wait_inputs((u + 1) % NB, (u + 1) % NBM)
                start_gather((u + 1) % NB, (u + 1) % NBM)
                wait_gather(u % NB, u % NBM)
                mul(u % NB)
                start_scatter(u % NB, u % NBM)
            return carry
        lax.fori_loop(1, (NCH - NBM) // NBM, step6, 0)

        for st in range(NCH - NBM, NCH):
            step(st)
        for st in range(NCH - 2, NCH):
            wait_scatter(st % NB, st % NBM)

        pltpu.sync_copy(tailm_hbm.at[wid], tailm_v)
        pltpu.sync_copy(w_hbm.at[pl.ds(ebase + NCH * E, ET)],
                        wv[0].at[pl.ds(0, ET)])
        pltpu.async_copy(f_hbm.at[tailm_v.at[0]],
                         rows_v[0].at[pl.ds(0, ET)], sem_g[0]).wait()

        def trow(r, cc):
            for j in range(D // 16):
                sl = pl.ds(j * 16, 16)
                rows_v[0][r, sl] = rows_v[0][r, sl] * wv[0][r, sl]
            return cc
        lax.fori_loop(0, ET, trow, 0)
        pltpu.sync_copy(rows_v[0].at[pl.ds(0, ET)],
                        conv_sh.at[tailm_v.at[1]], add=True)

        plsc.subcore_barrier()

        @pl.when(s < NS - 1)
        def _():
            for off, ln in RB_CH:
                src_off = s * ROWS_PER_TILE + off
                pltpu.sync_copy(conv_sh.at[pl.ds(src_off, ln)],
                                wv[0].at[pl.ds(0, ln)])
                pltpu.sync_copy(wv[0].at[pl.ds(0, ln)],
                                out_hbm.at[pl.ds(c * N_ATOMS + src_off, ln)])

        @pl.when(s == NS - 1)
        def _():
            for off, ln in RB_CH_LAST:
                src_off = s * ROWS_PER_TILE + off
                pltpu.sync_copy(conv_sh.at[pl.ds(src_off, ln)],
                                wv[0].at[pl.ds(0, ln)])
                pltpu.sync_copy(wv[0].at[pl.ds(0, ln)],
                                out_hbm.at[pl.ds(c * N_ATOMS + src_off, ln)])

    return sc_fn


_sc_kernel = _make_sc_kernel()


def kernel(x, w, seg_i, idx_j, W_in2fac, W_fac2out, b_fac2out):
    seg = seg_i.astype(jnp.int32)
    idx = idx_j.astype(jnp.int32)
    seg2 = seg.reshape(NW, EDGES_PER_W)
    idx2 = idx.reshape(NW, EDGES_PER_W)
    tailm = jnp.stack([idx2[:, NCH * E:], seg2[:, NCH * E:]], axis=1)

    f = pl.pallas_call(
        _mm1_body,
        out_shape=jax.ShapeDtypeStruct((N_ATOMS, D), jnp.float32),
    )(x, W_in2fac)

    parts = _sc_kernel(f, w, seg, idx, tailm)

    y = pl.pallas_call(
        _mm2_body,
        out_shape=jax.ShapeDtypeStruct((N_ATOMS, D), jnp.float32),
    )(parts, W_fac2out, b_fac2out.reshape(1, D))
    return y
